# Initial kernel scaffold; baseline (speedup 1.0000x reference)
import jax
import jax.numpy as jnp
from jax.experimental import pallas as pl


def kernel(x, edge_index, W1, b1, W2, b2, W3, b3):
    raise NotImplementedError("write your pallas kernel here")


# speedup vs baseline: 11.1707x; 11.1707x over previous
import functools

import jax
import jax.numpy as jnp
from jax import lax
from jax.experimental import pallas as pl
from jax.experimental.pallas import tpu as pltpu
import jax.experimental.pallas.tpu_sc as plsc

N = 10000
E = 320000
D = 128
DP3 = 16
N_PAD = 10240
NC = 2
NS = 16
NW = NC * NS
EPW = E // NW
K = 80
NB = EPW // K
RPS = N_PAD // NS

_mesh = plsc.VectorSubcoreMesh(core_axis_name="c", subcore_axis_name="s")


def _wid():
    return lax.axis_index("s") * NC + lax.axis_index("c")


@functools.partial(
    pl.kernel,
    out_type=jax.ShapeDtypeStruct((NC, N_PAD), jnp.float32),
    mesh=_mesh,
    scratch_types=[
        pltpu.VMEM((K,), jnp.int32),
        pltpu.VMEM((K,), jnp.float32),
        pltpu.VMEM_SHARED((N_PAD,), jnp.float32),
    ],
)
def _sc_count(dst_hbm, zeros_hbm, out_hbm, idx_v, ones_v, acc_sh):
    cid = lax.axis_index("c")
    sid = lax.axis_index("s")
    wid = sid * NC + cid

    def fill_ones(i, _):
        ones_v[pl.ds(i * 16, 16)] = jnp.ones((16,), jnp.float32)
        return 0

    lax.fori_loop(0, K // 16, fill_ones, 0)
    pltpu.sync_copy(zeros_hbm.at[pl.ds(sid * RPS, RPS)],
                    acc_sh.at[pl.ds(sid * RPS, RPS)])
    plsc.subcore_barrier()

    def body(i, _):
        base = wid * EPW + i * K
        pltpu.sync_copy(dst_hbm.at[pl.ds(base, K)], idx_v)
        pltpu.sync_copy(ones_v, acc_sh.at[idx_v], add=True)
        return 0

    lax.fori_loop(0, NB, body, 0)
    plsc.subcore_barrier()
    pltpu.sync_copy(acc_sh.at[pl.ds(sid * RPS, RPS)],
                    out_hbm.at[cid, pl.ds(sid * RPS, RPS)])


def _make_sc_agg(R):
    @functools.partial(
        pl.kernel,
        out_type=jax.ShapeDtypeStruct((NC, N_PAD, R), jnp.float32),
        mesh=_mesh,
        scratch_types=[
            pltpu.VMEM((K,), jnp.int32),
            pltpu.VMEM((K,), jnp.int32),
            pltpu.VMEM((K, R), jnp.float32),
            pltpu.SemaphoreType.DMA,
            pltpu.VMEM_SHARED((N_PAD, R), jnp.float32),
        ],
    )
    def _sc_agg(h_hbm, src_hbm, dst_hbm, zeros_hbm, out_hbm,
                src_v, dst_v, rows_v, gsem, acc_sh):
        cid = lax.axis_index("c")
        sid = lax.axis_index("s")
        wid = sid * NC + cid
        pltpu.sync_copy(zeros_hbm.at[pl.ds(sid * RPS, RPS)],
                        acc_sh.at[pl.ds(sid * RPS, RPS)])
        plsc.subcore_barrier()

        def body(i, _):
            base = wid * EPW + i * K
            pltpu.sync_copy(src_hbm.at[pl.ds(base, K)], src_v)
            pltpu.sync_copy(dst_hbm.at[pl.ds(base, K)], dst_v)
            pltpu.async_copy(h_hbm.at[src_v], rows_v, gsem).wait()
            pltpu.sync_copy(rows_v, acc_sh.at[dst_v], add=True)
            return 0

        lax.fori_loop(0, NB, body, 0)
        plsc.subcore_barrier()
        pltpu.sync_copy(acc_sh.at[pl.ds(sid * RPS, RPS)],
                        out_hbm.at[cid, pl.ds(sid * RPS, RPS)])

    return _sc_agg


_sc_agg128 = _make_sc_agg(D)


BN = 1024
GRID = N_PAD // BN

_row = lambda g: (g, 0)
_full = lambda g: (0, 0)


def _t12_body(x_ref, w_ref, cnt_ref, hs_ref, dinv_ref):
    dinv = lax.rsqrt(1.0 + cnt_ref[...])
    hw = jnp.dot(x_ref[...], w_ref[...], preferred_element_type=jnp.float32)
    hs_ref[...] = hw * dinv
    dinv_ref[...] = dinv


def _tc_stage1(x_pad, W1, cnt_col):
    return pl.pallas_call(
        _t12_body,
        grid=(GRID,),
        in_specs=[
            pl.BlockSpec((BN, D), _row),
            pl.BlockSpec((D, D), _full),
            pl.BlockSpec((BN, 1), _row),
        ],
        out_specs=[
            pl.BlockSpec((BN, D), _row),
            pl.BlockSpec((BN, 1), _row),
        ],
        out_shape=[
            jax.ShapeDtypeStruct((N_PAD, D), jnp.float32),
            jax.ShapeDtypeStruct((N_PAD, 1), jnp.float32),
        ],
    )(x_pad, W1, cnt_col)


def _tmid_body(a0_ref, a1_ref, hs_ref, dinv_ref, b_ref, w_ref, out_ref):
    dinv = dinv_ref[...]
    h = dinv * (a0_ref[...] + a1_ref[...] + hs_ref[...]) + b_ref[...]
    h = jnp.maximum(h, 0.0)
    hw = jnp.dot(h, w_ref[...], preferred_element_type=jnp.float32)
    out_ref[...] = hw * dinv


def _tc_mid(agg0, agg1, hs, dinv_col, b_row, W, r_out):
    return pl.pallas_call(
        _tmid_body,
        grid=(GRID,),
        in_specs=[
            pl.BlockSpec((BN, D), _row),
            pl.BlockSpec((BN, D), _row),
            pl.BlockSpec((BN, D), _row),
            pl.BlockSpec((BN, 1), _row),
            pl.BlockSpec((1, D), _full),
            pl.BlockSpec((D, r_out), _full),
        ],
        out_specs=pl.BlockSpec((BN, r_out), _row),
        out_shape=jax.ShapeDtypeStruct((N_PAD, r_out), jnp.float32),
    )(agg0, agg1, hs, dinv_col, b_row, W)


def _t4_body(a0_ref, a1_ref, hs_ref, dinv_ref, b_ref, out_ref):
    dinv = dinv_ref[...]
    h = dinv * (a0_ref[...] + a1_ref[...] + hs_ref[...]) + b_ref[...]
    out_ref[...] = dinv * jnp.maximum(h, 0.0)


def _tc_pre3(agg0, agg1, hs2, dinv_col, b2_row):
    return pl.pallas_call(
        _t4_body,
        grid=(GRID,),
        in_specs=[
            pl.BlockSpec((BN, D), _row),
            pl.BlockSpec((BN, D), _row),
            pl.BlockSpec((BN, D), _row),
            pl.BlockSpec((BN, 1), _row),
            pl.BlockSpec((1, D), _full),
        ],
        out_specs=pl.BlockSpec((BN, D), _row),
        out_shape=jax.ShapeDtypeStruct((N_PAD, D), jnp.float32),
    )(agg0, agg1, hs2, dinv_col, b2_row)


def _t5_body(a0_ref, a1_ref, g_ref, dinv_ref, w_ref, b_ref, out_ref):
    z = dinv_ref[...] * (a0_ref[...] + a1_ref[...] + g_ref[...])
    out_ref[...] = (jnp.dot(z, w_ref[...], preferred_element_type=jnp.float32)
                    + b_ref[...])


def _tc_final(agg0, agg1, g, dinv_col, W3p, b3_row):
    return pl.pallas_call(
        _t5_body,
        grid=(GRID,),
        in_specs=[
            pl.BlockSpec((BN, D), _row),
            pl.BlockSpec((BN, D), _row),
            pl.BlockSpec((BN, D), _row),
            pl.BlockSpec((BN, 1), _row),
            pl.BlockSpec((D, DP3), _full),
            pl.BlockSpec((1, DP3), _full),
        ],
        out_specs=pl.BlockSpec((BN, DP3), _row),
        out_shape=jax.ShapeDtypeStruct((N_PAD, DP3), jnp.float32),
    )(agg0, agg1, g, dinv_col, W3p, b3_row)


def kernel(x, edge_index, W1, b1, W2, b2, W3, b3):
    x_pad = jnp.pad(x, ((0, N_PAD - N), (0, 0)))
    W3p = jnp.pad(W3, ((0, 0), (0, DP3 - W3.shape[1])))
    b1r = b1.reshape(1, D)
    b2r = b2.reshape(1, D)
    b3r = jnp.pad(b3, (0, DP3 - b3.shape[0])).reshape(1, DP3)
    z1 = jnp.zeros((N_PAD,), jnp.float32)
    z128 = jnp.zeros((N_PAD, D), jnp.float32)

    src = edge_index[0]
    dst = edge_index[1]

    cnt_parts = _sc_count(dst, z1)
    cnt_col = (cnt_parts[0] + cnt_parts[1]).reshape(N_PAD, 1)

    hs1, dinv_col = _tc_stage1(x_pad, W1, cnt_col)

    agg1 = _sc_agg128(hs1, src, dst, z128)
    hs2 = _tc_mid(agg1[0], agg1[1], hs1, dinv_col, b1r, W2, D)

    agg2 = _sc_agg128(hs2, src, dst, z128)
    g = _tc_pre3(agg2[0], agg2[1], hs2, dinv_col, b2r)

    agg3 = _sc_agg128(g, src, dst, z128)
    out16 = _tc_final(agg3[0], agg3[1], g, dinv_col, W3p, b3r)

    return out16[:N, :2]


# speedup vs baseline: 30.1898x; 2.7026x over previous
import functools

import jax
import jax.numpy as jnp
from jax import lax
from jax.experimental import pallas as pl
from jax.experimental.pallas import tpu as pltpu
import jax.experimental.pallas.tpu_sc as plsc

N = 10000
E = 320000
D = 128
DP3 = 16
N_PAD = 10240
NC = 2
NS = 16
NW = NC * NS
EPW = E // NW
K = 80
NB = EPW // K
RPS = N_PAD // NS

_mesh = plsc.VectorSubcoreMesh(core_axis_name="c", subcore_axis_name="s")


def _wid():
    return lax.axis_index("s") * NC + lax.axis_index("c")


NBUF = 5
NRING = NB // NBUF - 1


NBUF = 5
NRING = NB // NBUF - 1


@functools.partial(
    pl.kernel,
    out_type=jax.ShapeDtypeStruct((NC, N_PAD), jnp.float32),
    mesh=_mesh,
    scratch_types=[
        pltpu.VMEM((NBUF, K), jnp.int32),
        pltpu.VMEM((K,), jnp.float32),
        pltpu.SemaphoreType.DMA((NBUF,)),
        pltpu.VMEM_SHARED((N_PAD,), jnp.float32),
    ],
)
def _sc_count(dst_hbm, zeros_hbm, out_hbm, dst_v, ones_v, isem, acc_sh):
    cid = lax.axis_index("c")
    sid = lax.axis_index("s")
    wid = sid * NC + cid
    base0 = wid * EPW

    def fill_ones(i, _):
        ones_v[pl.ds(i * 16, 16)] = jnp.ones((16,), jnp.float32)
        return 0

    lax.fori_loop(0, K // 16, fill_ones, 0)
    pltpu.sync_copy(zeros_hbm.at[pl.ds(sid * RPS, RPS)],
                    acc_sh.at[pl.ds(sid * RPS, RPS)])
    plsc.subcore_barrier()

    def fire(j, b):
        pltpu.async_copy(dst_hbm.at[pl.ds(base0 + j * K, K)],
                         dst_v.at[b], isem.at[b])

    def drain_and_scatter(j, b):
        pltpu.make_async_copy(dst_hbm.at[pl.ds(base0 + j * K, K)],
                              dst_v.at[b], isem.at[b]).wait()
        pltpu.sync_copy(ones_v, acc_sh.at[dst_v.at[b]], add=True)

    for b in range(NBUF):
        fire(b, b)

    def body(g, _):
        for b in range(NBUF):
            j = g * NBUF + b
            drain_and_scatter(j, b)
            fire(j + NBUF, b)
        return 0

    lax.fori_loop(0, NRING, body, 0)
    for b in range(NBUF):
        drain_and_scatter(NRING * NBUF + b, b)
    plsc.subcore_barrier()
    pltpu.sync_copy(acc_sh.at[pl.ds(sid * RPS, RPS)],
                    out_hbm.at[cid, pl.ds(sid * RPS, RPS)])


KA = 40
NBA = EPW // KA
NRINGA = NBA // NBUF - 1


def _make_sc_agg(R):
    @functools.partial(
        pl.kernel,
        out_type=jax.ShapeDtypeStruct((NC, N_PAD, R), jnp.float32),
        mesh=_mesh,
        scratch_types=[
            pltpu.VMEM((EPW,), jnp.int32),
            pltpu.VMEM((NBUF, KA), jnp.int32),
            pltpu.VMEM((NBUF, KA, R), jnp.float32),
            pltpu.SemaphoreType.DMA((NBUF,)),
            pltpu.SemaphoreType.DMA((NBUF,)),
            pltpu.VMEM_SHARED((N_PAD, R), jnp.float32),
        ],
    )
    def _sc_agg(h_hbm, src_hbm, dst_hbm, zeros_hbm, out_hbm,
                src_all, dst_v, rows_v, isem, gsem, acc_sh):
        cid = lax.axis_index("c")
        sid = lax.axis_index("s")
        wid = sid * NC + cid
        base0 = wid * EPW
        pltpu.sync_copy(zeros_hbm.at[pl.ds(sid * RPS, RPS)],
                        acc_sh.at[pl.ds(sid * RPS, RPS)])
        pltpu.sync_copy(src_hbm.at[pl.ds(base0, EPW)], src_all)
        plsc.subcore_barrier()

        def fire(j, b):
            pltpu.async_copy(dst_hbm.at[pl.ds(base0 + j * KA, KA)],
                             dst_v.at[b], isem.at[b])
            pltpu.async_copy(h_hbm.at[src_all.at[pl.ds(j * KA, KA)]],
                             rows_v.at[b], gsem.at[b])

        def drain_and_scatter(j, b):
            pltpu.make_async_copy(h_hbm.at[src_all.at[pl.ds(j * KA, KA)]],
                                  rows_v.at[b], gsem.at[b]).wait()
            pltpu.make_async_copy(dst_hbm.at[pl.ds(base0 + j * KA, KA)],
                                  dst_v.at[b], isem.at[b]).wait()
            pltpu.sync_copy(rows_v.at[b], acc_sh.at[dst_v.at[b]], add=True)

        for b in range(NBUF):
            fire(b, b)

        def body(g, _):
            for b in range(NBUF):
                j = g * NBUF + b
                drain_and_scatter(j, b)
                fire(j + NBUF, b)
            return 0

        lax.fori_loop(0, NRINGA, body, 0)
        for b in range(NBUF):
            drain_and_scatter(NRINGA * NBUF + b, b)
        plsc.subcore_barrier()
        pltpu.sync_copy(acc_sh.at[pl.ds(sid * RPS, RPS)],
                        out_hbm.at[cid, pl.ds(sid * RPS, RPS)])

    return _sc_agg


_sc_agg128 = _make_sc_agg(D)


BN = 1024
GRID = N_PAD // BN

_row = lambda g: (g, 0)
_full = lambda g: (0, 0)


def _t12_body(x_ref, w_ref, cnt_ref, hs_ref, dinv_ref):
    dinv = lax.rsqrt(1.0 + cnt_ref[...])
    hw = jnp.dot(x_ref[...], w_ref[...], preferred_element_type=jnp.float32)
    hs_ref[...] = hw * dinv
    dinv_ref[...] = dinv


def _tc_stage1(x_pad, W1, cnt_col):
    return pl.pallas_call(
        _t12_body,
        grid=(GRID,),
        in_specs=[
            pl.BlockSpec((BN, D), _row),
            pl.BlockSpec((D, D), _full),
            pl.BlockSpec((BN, 1), _row),
        ],
        out_specs=[
            pl.BlockSpec((BN, D), _row),
            pl.BlockSpec((BN, 1), _row),
        ],
        out_shape=[
            jax.ShapeDtypeStruct((N_PAD, D), jnp.float32),
            jax.ShapeDtypeStruct((N_PAD, 1), jnp.float32),
        ],
    )(x_pad, W1, cnt_col)


def _tmid_body(a0_ref, a1_ref, hs_ref, dinv_ref, b_ref, w_ref, out_ref):
    dinv = dinv_ref[...]
    h = dinv * (a0_ref[...] + a1_ref[...] + hs_ref[...]) + b_ref[...]
    h = jnp.maximum(h, 0.0)
    hw = jnp.dot(h, w_ref[...], preferred_element_type=jnp.float32)
    out_ref[...] = hw * dinv


def _tc_mid(agg0, agg1, hs, dinv_col, b_row, W, r_out):
    return pl.pallas_call(
        _tmid_body,
        grid=(GRID,),
        in_specs=[
            pl.BlockSpec((BN, D), _row),
            pl.BlockSpec((BN, D), _row),
            pl.BlockSpec((BN, D), _row),
            pl.BlockSpec((BN, 1), _row),
            pl.BlockSpec((1, D), _full),
            pl.BlockSpec((D, r_out), _full),
        ],
        out_specs=pl.BlockSpec((BN, r_out), _row),
        out_shape=jax.ShapeDtypeStruct((N_PAD, r_out), jnp.float32),
    )(agg0, agg1, hs, dinv_col, b_row, W)


def _t4_body(a0_ref, a1_ref, hs_ref, dinv_ref, b_ref, out_ref):
    dinv = dinv_ref[...]
    h = dinv * (a0_ref[...] + a1_ref[...] + hs_ref[...]) + b_ref[...]
    out_ref[...] = dinv * jnp.maximum(h, 0.0)


def _tc_pre3(agg0, agg1, hs2, dinv_col, b2_row):
    return pl.pallas_call(
        _t4_body,
        grid=(GRID,),
        in_specs=[
            pl.BlockSpec((BN, D), _row),
            pl.BlockSpec((BN, D), _row),
            pl.BlockSpec((BN, D), _row),
            pl.BlockSpec((BN, 1), _row),
            pl.BlockSpec((1, D), _full),
        ],
        out_specs=pl.BlockSpec((BN, D), _row),
        out_shape=jax.ShapeDtypeStruct((N_PAD, D), jnp.float32),
    )(agg0, agg1, hs2, dinv_col, b2_row)


def _t5_body(a0_ref, a1_ref, g_ref, dinv_ref, w_ref, b_ref, out_ref):
    z = dinv_ref[...] * (a0_ref[...] + a1_ref[...] + g_ref[...])
    out_ref[...] = (jnp.dot(z, w_ref[...], preferred_element_type=jnp.float32)
                    + b_ref[...])


def _tc_final(agg0, agg1, g, dinv_col, W3p, b3_row):
    return pl.pallas_call(
        _t5_body,
        grid=(GRID,),
        in_specs=[
            pl.BlockSpec((BN, D), _row),
            pl.BlockSpec((BN, D), _row),
            pl.BlockSpec((BN, D), _row),
            pl.BlockSpec((BN, 1), _row),
            pl.BlockSpec((D, DP3), _full),
            pl.BlockSpec((1, DP3), _full),
        ],
        out_specs=pl.BlockSpec((BN, DP3), _row),
        out_shape=jax.ShapeDtypeStruct((N_PAD, DP3), jnp.float32),
    )(agg0, agg1, g, dinv_col, W3p, b3_row)


def kernel(x, edge_index, W1, b1, W2, b2, W3, b3):
    x_pad = jnp.pad(x, ((0, N_PAD - N), (0, 0)))
    W3p = jnp.pad(W3, ((0, 0), (0, DP3 - W3.shape[1])))
    b1r = b1.reshape(1, D)
    b2r = b2.reshape(1, D)
    b3r = jnp.pad(b3, (0, DP3 - b3.shape[0])).reshape(1, DP3)
    z1 = jnp.zeros((N_PAD,), jnp.float32)
    z128 = jnp.zeros((N_PAD, D), jnp.float32)

    src = edge_index[0]
    dst = edge_index[1]

    cnt_parts = _sc_count(dst, z1)
    cnt_col = (cnt_parts[0] + cnt_parts[1]).reshape(N_PAD, 1)

    hs1, dinv_col = _tc_stage1(x_pad, W1, cnt_col)

    agg1 = _sc_agg128(hs1, src, dst, z128)
    hs2 = _tc_mid(agg1[0], agg1[1], hs1, dinv_col, b1r, W2, D)

    agg2 = _sc_agg128(hs2, src, dst, z128)
    g = _tc_pre3(agg2[0], agg2[1], hs2, dinv_col, b2r)

    agg3 = _sc_agg128(g, src, dst, z128)
    out16 = _tc_final(agg3[0], agg3[1], g, dinv_col, W3p, b3r)

    return out16[:N, :2]
